# Initial kernel scaffold; baseline (speedup 1.0000x reference)
import jax
import jax.numpy as jnp
from jax.experimental import pallas as pl


def kernel(features, W, gamma, beta):
    raise NotImplementedError("write your pallas kernel here")


# speedup vs baseline: 2.8346x; 2.8346x over previous
import functools

import jax
import jax.numpy as jnp
from jax.experimental import pallas as pl

B, C, N, K = 8, 16, 2048, 20
D_OUT = 64
EPS = 1e-5
TN = 256
BIG = 3.0e38


def _knn_body(pts_ref, sqr_ref, sqc_ref, w_ref, y_ref, sums_ref):
    b = pl.program_id(0)
    t = pl.program_id(1)

    pts = pts_ref[0]
    xt = pts_ref[0, pl.ds(t * TN, TN), :]

    sq_t = sqc_ref[0, pl.ds(t * TN, TN), :]
    sq_row = sqr_ref[0]
    prod = jax.lax.dot_general(
        xt, pts, (((1,), (1,)), ((), ())),
        preferred_element_type=jnp.float32)
    d = (sq_t + sq_row) - 2.0 * prod

    w = w_ref[...]
    col = jax.lax.broadcasted_iota(jnp.int32, (TN, N), 1)

    s_acc = jnp.zeros((1, D_OUT), jnp.float32)
    q_acc = jnp.zeros((1, D_OUT), jnp.float32)

    def step(j, carry):
        d, s_acc, q_acc = carry
        m = jnp.min(d, axis=1, keepdims=True)
        idxf = jnp.min(jnp.where(d == m, col, N), axis=1, keepdims=True)
        onehot = (col == idxf).astype(jnp.float32)
        feat = jax.lax.dot_general(
            onehot, pts, (((1,), (0,)), ((), ())),
            preferred_element_type=jnp.float32,
            precision=jax.lax.Precision.HIGHEST)
        f = jnp.concatenate(
            [xt, feat, feat - xt, jnp.broadcast_to(m, (TN, C))],
            axis=1)
        y = jax.lax.dot_general(
            f, w, (((1,), (1,)), ((), ())),
            preferred_element_type=jnp.float32)
        y_ref[0, j, :, :] = y
        s_acc = s_acc + jnp.sum(y, axis=0, keepdims=True)
        q_acc = q_acc + jnp.sum(y * y, axis=0, keepdims=True)
        d = jnp.where(onehot > 0.0, BIG, d)
        return d, s_acc, q_acc

    d, s_acc, q_acc = jax.lax.fori_loop(0, K, step, (d, s_acc, q_acc))

    @pl.when(jnp.logical_and(b == 0, t == 0))
    def _():
        sums_ref[...] = jnp.zeros_like(sums_ref)

    sums_ref[0:1, :] += s_acc
    sums_ref[1:2, :] += q_acc


def _norm_body(y_ref, sc_ref, out_ref):
    z = sc_ref[0:1, :] * y_ref[0] + sc_ref[1:2, :]
    out_ref[0] = jnp.where(z >= 0.0, z, 0.2 * z)


def kernel(features, W, gamma, beta):
    x = jnp.squeeze(features, axis=-1)
    pts = jnp.transpose(x, (0, 2, 1))

    sq = jnp.sum(pts * pts, axis=-1)
    sqr = sq[:, None, :]
    sqc = sq[:, :, None]

    grid = (B, N // TN)
    y, sums = pl.pallas_call(
        _knn_body,
        grid=grid,
        in_specs=[
            pl.BlockSpec((1, N, C), lambda b, t: (b, 0, 0)),
            pl.BlockSpec((1, 1, N), lambda b, t: (b, 0, 0)),
            pl.BlockSpec((1, N, 1), lambda b, t: (b, 0, 0)),
            pl.BlockSpec((D_OUT, 4 * C), lambda b, t: (0, 0)),
        ],
        out_specs=[
            pl.BlockSpec((1, K, TN, D_OUT), lambda b, t: (b, 0, t, 0)),
            pl.BlockSpec((2, D_OUT), lambda b, t: (0, 0)),
        ],
        out_shape=[
            jax.ShapeDtypeStruct((B, K, N, D_OUT), jnp.float32),
            jax.ShapeDtypeStruct((2, D_OUT), jnp.float32),
        ],
    )(pts, sqr, sqc, W)

    cnt = jnp.float32(B * N * K)
    mean = sums[0] / cnt
    var = sums[1] / cnt - mean * mean
    scale = gamma / jnp.sqrt(var + EPS)
    bias = beta - scale * mean
    sb = jnp.stack([scale, bias], axis=0)

    z = pl.pallas_call(
        _norm_body,
        grid=(B, K),
        in_specs=[
            pl.BlockSpec((1, 1, N, D_OUT), lambda b, j: (b, j, 0, 0)),
            pl.BlockSpec((2, D_OUT), lambda b, j: (0, 0)),
        ],
        out_specs=pl.BlockSpec((1, 1, N, D_OUT), lambda b, j: (b, j, 0, 0)),
        out_shape=jax.ShapeDtypeStruct((B, K, N, D_OUT), jnp.float32),
    )(y, sb)

    return jnp.transpose(z, (0, 3, 2, 1))


# speedup vs baseline: 5.5802x; 1.9686x over previous
import functools

import jax
import jax.numpy as jnp
from jax import lax
from jax.experimental import pallas as pl
from jax.experimental.pallas import tpu as pltpu
from jax.experimental.pallas import tpu_sc as plsc

B, C, N, K = 8, 16, 2048, 20
D_OUT = 64
EPS = 1e-5
BIG = 3.0e38

R = B * N
NW = 32
RPW = R // NW
RB = 8
NBLK = RPW // RB
CAND = 64
CPAD = 96
NCH = N // 16

TN = 256
TS = 512
GPW = (R * K) // NW
GCH = 128


def _dist_body(ptsb_ref, xt_ref, sqr_ref, sqc_ref, d_ref):
    pts = ptsb_ref[0]
    xt = xt_ref[...]
    prod = jax.lax.dot_general(
        xt, pts, (((1,), (1,)), ((), ())),
        preferred_element_type=jnp.float32)
    d_ref[...] = (sqc_ref[...] + sqr_ref[0]) - 2.0 * prod


def _sc_filter_body(dmat, candv_hbm, candi_hbm, buf, ov, oi, iota_f):
    wid = lax.axis_index("s") * 2 + lax.axis_index("c")
    iota = lax.iota(jnp.int32, 16)
    iota_f[...] = iota.astype(jnp.float32)

    def block(blk, _):
        row0 = wid * RPW + blk * RB
        pltpu.sync_copy(dmat.at[pl.ds(row0, RB), :], buf)

        for r in range(RB):
            def p1(c, carry):
                m1, m2 = carry
                v = buf[r, pl.ds(c * 16, 16)]
                a = jnp.minimum(m1, v)
                bmx = jnp.maximum(m1, v)
                return a, jnp.minimum(m2, bmx)

            m1, m2 = lax.fori_loop(
                0, NCH, p1,
                (jnp.full((16,), BIG, jnp.float32),
                 jnp.full((16,), BIG, jnp.float32)))

            s1, _ = plsc.sort_key_val(m1, m1)
            s2, _ = plsc.sort_key_val(m2, m2)
            s2r = lax.rev(s2, (0,))
            hi, _ = plsc.sort_key_val(jnp.maximum(s1, s2r),
                                      jnp.maximum(s1, s2r))
            t = jnp.min(jnp.where(iota == 3, hi, BIG))

            for q in range(CPAD // 16):
                ov[pl.ds(r * CPAD + q * 16, 16)] = jnp.full(
                    (16,), BIG, jnp.float32)
                oi[pl.ds(r * CPAD + q * 16, 16)] = jnp.zeros(
                    (16,), jnp.float32)

            def flt(c, cnt):
                v = buf[r, pl.ds(c * 16, 16)]
                mask = v <= t
                idxf = (c * 16).astype(jnp.float32) + iota_f[...]
                off = r * CPAD + jnp.minimum(cnt, CAND)
                plsc.store_compressed(ov.at[pl.ds(off, 16)], v, mask=mask)
                plsc.store_compressed(oi.at[pl.ds(off, 16)], idxf, mask=mask)
                return cnt + jnp.sum(mask.astype(jnp.int32))

            lax.fori_loop(0, NCH, flt, jnp.int32(0))

            pltpu.sync_copy(
                ov.at[pl.ds(r * CPAD, CAND)],
                candv_hbm.at[pl.ds((row0 + r) * CAND, CAND)])
            pltpu.sync_copy(
                oi.at[pl.ds(r * CPAD, CAND)],
                candi_hbm.at[pl.ds((row0 + r) * CAND, CAND)])
        return 0

    lax.fori_loop(0, NBLK, block, 0)


def _select_body(cv_ref, ci_ref, gidx_ref, dist_ref):
    b = pl.program_id(0)
    cv = cv_ref[0, 0]
    ci = ci_ref[0, 0]
    col = jax.lax.broadcasted_iota(jnp.int32, (TS, CAND), 1)
    gs, ms = [], []
    for j in range(K):
        m = jnp.min(cv, axis=1, keepdims=True)
        sel = jnp.min(jnp.where(cv == m, col, CAND), axis=1, keepdims=True)
        onehot = col == sel
        gi = jnp.min(jnp.where(onehot, ci, BIG), axis=1, keepdims=True)
        gs.append(gi)
        ms.append(m)
        cv = jnp.where(onehot, BIG, cv)
    del b
    gidx = jnp.concatenate(gs, axis=1)
    gidx_ref[0, 0] = gidx.astype(jnp.int32)
    dist_ref[0, 0] = jnp.concatenate(ms, axis=1)


def _sc_gather_body(gidx_hbm, pts_hbm, feat_hbm, ptbuf, idx_v, out_v):
    wid = lax.axis_index("s") * 2 + lax.axis_index("c")
    b = wid // (NW // B)
    pltpu.sync_copy(pts_hbm.at[pl.ds(b * (N * C), N * C)], ptbuf)
    iota = lax.iota(jnp.int32, 16)

    def chunk(i, _):
        c0 = wid * GPW + i * GCH
        pltpu.sync_copy(gidx_hbm.at[pl.ds(c0, GCH)], idx_v)
        for g in range(GCH // 16):
            base = idx_v[pl.ds(g * 16, 16)] * C
            dbase = g * (16 * C) + iota * C
            for c in range(C):
                vals = plsc.load_gather(ptbuf, [base + c])
                plsc.store_scatter(out_v, [dbase + c], vals)
        pltpu.sync_copy(out_v, feat_hbm.at[pl.ds(c0 * C, GCH * C)])
        return 0

    lax.fori_loop(0, GPW // GCH, chunk, 0)


def _conv_body(xt_ref, feat_ref, dist_ref, w_ref, y_ref, sums_ref):
    t = pl.program_id(0)
    xt = xt_ref[...]
    w = w_ref[...]
    s_acc = jnp.zeros((1, D_OUT), jnp.float32)
    q_acc = jnp.zeros((1, D_OUT), jnp.float32)
    for j in range(K):
        feat = feat_ref[:, j * C:(j + 1) * C]
        m = dist_ref[:, j:j + 1]
        f = jnp.concatenate(
            [xt, feat, feat - xt, jnp.broadcast_to(m, (TS, C))], axis=1)
        y = jax.lax.dot_general(
            f, w, (((1,), (1,)), ((), ())),
            preferred_element_type=jnp.float32)
        y_ref[j] = y
        s_acc = s_acc + jnp.sum(y, axis=0, keepdims=True)
        q_acc = q_acc + jnp.sum(y * y, axis=0, keepdims=True)

    @pl.when(t == 0)
    def _():
        sums_ref[...] = jnp.zeros_like(sums_ref)

    sums_ref[0:1, :] += s_acc
    sums_ref[1:2, :] += q_acc


def _norm_body(y_ref, sc_ref, out_ref):
    z = sc_ref[0:1, :] * y_ref[0] + sc_ref[1:2, :]
    out_ref[0] = jnp.where(z >= 0.0, z, 0.2 * z)


def kernel(features, W, gamma, beta):
    x = jnp.squeeze(features, axis=-1)
    pts = jnp.transpose(x, (0, 2, 1))
    pts_flat = pts.reshape(R, C)
    sq = jnp.sum(pts * pts, axis=-1)
    sqr = sq[:, None, :]
    sqc = sq.reshape(R, 1)

    dmat = pl.pallas_call(
        _dist_body,
        grid=(R // TN,),
        in_specs=[
            pl.BlockSpec((1, N, C), lambda t: (t // (N // TN), 0, 0)),
            pl.BlockSpec((TN, C), lambda t: (t, 0)),
            pl.BlockSpec((1, 1, N), lambda t: (t // (N // TN), 0, 0)),
            pl.BlockSpec((TN, 1), lambda t: (t, 0)),
        ],
        out_specs=pl.BlockSpec((TN, N), lambda t: (t, 0)),
        out_shape=jax.ShapeDtypeStruct((R, N), jnp.float32),
    )(pts, pts_flat, sqr, sqc)

    mesh = plsc.VectorSubcoreMesh(core_axis_name="c", subcore_axis_name="s")
    candv, candi = pl.kernel(
        _sc_filter_body,
        mesh=mesh,
        compiler_params=pltpu.CompilerParams(needs_layout_passes=False),
        out_type=[
            jax.ShapeDtypeStruct((R * CAND,), jnp.float32),
            jax.ShapeDtypeStruct((R * CAND,), jnp.float32),
        ],
        scratch_types=[
            pltpu.VMEM((RB, N), jnp.float32),
            pltpu.VMEM((RB * CPAD,), jnp.float32),
            pltpu.VMEM((RB * CPAD,), jnp.float32),
            pltpu.VMEM((16,), jnp.float32),
        ],
    )(dmat)

    gidx, dist = pl.pallas_call(
        _select_body,
        grid=(B, N // TS),
        in_specs=[
            pl.BlockSpec((1, 1, TS, CAND), lambda b, t: (b, t, 0, 0)),
            pl.BlockSpec((1, 1, TS, CAND), lambda b, t: (b, t, 0, 0)),
        ],
        out_specs=[
            pl.BlockSpec((1, 1, TS, K), lambda b, t: (b, t, 0, 0)),
            pl.BlockSpec((1, 1, TS, K), lambda b, t: (b, t, 0, 0)),
        ],
        out_shape=[
            jax.ShapeDtypeStruct((B, N // TS, TS, K), jnp.int32),
            jax.ShapeDtypeStruct((B, N // TS, TS, K), jnp.float32),
        ],
    )(candv.reshape(B, N // TS, TS, CAND),
      candi.reshape(B, N // TS, TS, CAND))
    gidx_flat = gidx.reshape(R * K)
    dist_flat = dist.reshape(R, K)

    feat = pl.kernel(
        _sc_gather_body,
        mesh=mesh,
        compiler_params=pltpu.CompilerParams(needs_layout_passes=False),
        out_type=jax.ShapeDtypeStruct((R * K * C,), jnp.float32),
        scratch_types=[
            pltpu.VMEM((N * C,), jnp.float32),
            pltpu.VMEM((GCH,), jnp.int32),
            pltpu.VMEM((GCH * C,), jnp.float32),
        ],
    )(gidx_flat, pts_flat.reshape(R * C))

    y, sums = pl.pallas_call(
        _conv_body,
        grid=(R // TS,),
        in_specs=[
            pl.BlockSpec((TS, C), lambda t: (t, 0)),
            pl.BlockSpec((TS, K * C), lambda t: (t, 0)),
            pl.BlockSpec((TS, K), lambda t: (t, 0)),
            pl.BlockSpec((D_OUT, 4 * C), lambda t: (0, 0)),
        ],
        out_specs=[
            pl.BlockSpec((K, TS, D_OUT), lambda t: (0, t, 0)),
            pl.BlockSpec((2, D_OUT), lambda t: (0, 0)),
        ],
        out_shape=[
            jax.ShapeDtypeStruct((K, R, D_OUT), jnp.float32),
            jax.ShapeDtypeStruct((2, D_OUT), jnp.float32),
        ],
    )(pts_flat, feat.reshape(R, K * C), dist_flat, W)

    cnt = jnp.float32(R * K)
    mean = sums[0] / cnt
    var = sums[1] / cnt - mean * mean
    scale = gamma / jnp.sqrt(var + EPS)
    bias = beta - scale * mean
    sb = jnp.stack([scale, bias], axis=0)

    z = pl.pallas_call(
        _norm_body,
        grid=(K, R // 1024),
        in_specs=[
            pl.BlockSpec((1, 1024, D_OUT), lambda j, t: (j, t, 0)),
            pl.BlockSpec((2, D_OUT), lambda j, t: (0, 0)),
        ],
        out_specs=pl.BlockSpec((1, 1024, D_OUT), lambda j, t: (j, t, 0)),
        out_shape=jax.ShapeDtypeStruct((K, R, D_OUT), jnp.float32),
    )(y, sb)

    return z.reshape(K, B, N, D_OUT).transpose(1, 3, 2, 0)


# speedup vs baseline: 7.1281x; 1.2774x over previous
import functools

import jax
import jax.numpy as jnp
from jax import lax
from jax.experimental import pallas as pl
from jax.experimental.pallas import tpu as pltpu
from jax.experimental.pallas import tpu_sc as plsc

B, C, N, K = 8, 16, 2048, 20
D_OUT = 64
EPS = 1e-5
BIG = 3.0e38

R = B * N
NW = 32
RPW = R // NW
RB = 8
NBLK = RPW // RB
CAND = 64
CPAD = 96
NCH = N // 16

TN = 256
TS = 512
GPW = (R * K) // NW
GCH = 128


def _dist_body(ptsb_ref, xt_ref, sqr_ref, sqc_ref, d_ref):
    pts = ptsb_ref[0]
    xt = xt_ref[...]
    prod = jax.lax.dot_general(
        xt, pts, (((1,), (1,)), ((), ())),
        preferred_element_type=jnp.float32)
    d_ref[...] = (sqc_ref[...] + sqr_ref[0]) - 2.0 * prod


def _sc_filter_body(dmat, candv_hbm, candi_hbm, buf, oi, pkv, pki):
    wid = lax.axis_index("s") * 2 + lax.axis_index("c")
    iota = lax.iota(jnp.int32, 16)
    iota_f = iota.astype(jnp.float32)
    bigv = jnp.full((16,), BIG, jnp.float32)

    for q in range(RB * CPAD // 16):
        oi[pl.ds(q * 16, 16)] = jnp.zeros((16,), jnp.float32)

    def block(blk, _):
        row0 = wid * RPW + blk * RB
        pltpu.sync_copy(dmat.at[pl.ds(row0, RB), :], buf)

        def p1(c, carry):
            out = []
            for r in range(RB):
                m1, m2 = carry[2 * r], carry[2 * r + 1]
                v = buf[r, pl.ds(c * 16, 16)]
                a = jnp.minimum(m1, v)
                bmx = jnp.maximum(m1, v)
                out += [a, jnp.minimum(m2, bmx)]
            return tuple(out)

        accs = lax.fori_loop(0, NCH, p1, (bigv,) * (2 * RB), unroll=4)

        ts = []
        for r in range(RB):
            m1, m2 = accs[2 * r], accs[2 * r + 1]
            s1, _ = plsc.sort_key_val(m1, m1)
            s2, _ = plsc.sort_key_val(m2, m2)
            hi = jnp.maximum(s1, lax.rev(s2, (0,)))
            shi, _ = plsc.sort_key_val(hi, hi)
            ts.append(jnp.min(jnp.where(iota == 3, shi, BIG)))

        def flt(c, cnts):
            new = []
            idxf = (c * 16).astype(jnp.float32) + iota_f
            for r in range(RB):
                v = buf[r, pl.ds(c * 16, 16)]
                mask = v <= ts[r]
                off = r * CPAD + jnp.minimum(cnts[r], CAND)
                plsc.store_compressed(oi.at[pl.ds(off, 16)], idxf, mask=mask)
                new.append(cnts[r] + jnp.sum(mask.astype(jnp.int32)))
            return tuple(new)

        cnts = lax.fori_loop(0, NCH, flt, (jnp.int32(0),) * RB, unroll=2)

        for r in range(RB):
            rsplat = jnp.full((16,), r, jnp.int32)
            for q in range(CAND // 16):
                lane = q * 16 + iota
                idxf = oi[pl.ds(r * CPAD + q * 16, 16)]
                vals = plsc.load_gather(buf, [rsplat, idxf.astype(jnp.int32)])
                vals = jnp.where(lane < cnts[r], vals, bigv)
                pkv[pl.ds(r * CAND + q * 16, 16)] = vals
                pki[pl.ds(r * CAND + q * 16, 16)] = idxf

        pltpu.sync_copy(pkv, candv_hbm.at[pl.ds(row0 * CAND, RB * CAND)])
        pltpu.sync_copy(pki, candi_hbm.at[pl.ds(row0 * CAND, RB * CAND)])
        return 0

    lax.fori_loop(0, NBLK, block, 0)


def _select_body(cv_ref, ci_ref, gidx_ref, dist_ref):
    b = pl.program_id(0)
    cv = cv_ref[0, 0]
    ci = ci_ref[0, 0]
    col = jax.lax.broadcasted_iota(jnp.int32, (TS, CAND), 1)
    gs, ms = [], []
    for j in range(K):
        m = jnp.min(cv, axis=1, keepdims=True)
        sel = jnp.min(jnp.where(cv == m, col, CAND), axis=1, keepdims=True)
        onehot = col == sel
        gi = jnp.min(jnp.where(onehot, ci, BIG), axis=1, keepdims=True)
        gs.append(gi)
        ms.append(m)
        cv = jnp.where(onehot, BIG, cv)
    del b
    gidx = jnp.concatenate(gs, axis=1)
    gidx_ref[0, 0] = gidx.astype(jnp.int32)
    dist_ref[0, 0] = jnp.concatenate(ms, axis=1)


def _sc_gather_body(gidx_hbm, pts_hbm, feat_hbm, ptbuf, idx_v, out_v):
    wid = lax.axis_index("s") * 2 + lax.axis_index("c")
    b = wid // (NW // B)
    pltpu.sync_copy(pts_hbm.at[pl.ds(b * (N * C), N * C)], ptbuf)
    iota = lax.iota(jnp.int32, 16)

    def chunk(i, _):
        c0 = wid * GPW + i * GCH
        pltpu.sync_copy(gidx_hbm.at[pl.ds(c0, GCH)], idx_v)
        for g in range(GCH // 16):
            base = idx_v[pl.ds(g * 16, 16)] * C
            dbase = g * (16 * C) + iota * C
            for c in range(C):
                vals = plsc.load_gather(ptbuf, [base + c])
                plsc.store_scatter(out_v, [dbase + c], vals)
        pltpu.sync_copy(out_v, feat_hbm.at[pl.ds(c0 * C, GCH * C)])
        return 0

    lax.fori_loop(0, GPW // GCH, chunk, 0)


def _conv_body(xt_ref, feat_ref, dist_ref, w_ref, y_ref, sums_ref):
    t = pl.program_id(0)
    xt = xt_ref[...]
    w = w_ref[...]
    s_acc = jnp.zeros((1, D_OUT), jnp.float32)
    q_acc = jnp.zeros((1, D_OUT), jnp.float32)
    for j in range(K):
        feat = feat_ref[:, j * C:(j + 1) * C]
        m = dist_ref[:, j:j + 1]
        f = jnp.concatenate(
            [xt, feat, feat - xt, jnp.broadcast_to(m, (TS, C))], axis=1)
        y = jax.lax.dot_general(
            f, w, (((1,), (1,)), ((), ())),
            preferred_element_type=jnp.float32)
        y_ref[j] = y
        s_acc = s_acc + jnp.sum(y, axis=0, keepdims=True)
        q_acc = q_acc + jnp.sum(y * y, axis=0, keepdims=True)

    @pl.when(t == 0)
    def _():
        sums_ref[...] = jnp.zeros_like(sums_ref)

    sums_ref[0:1, :] += s_acc
    sums_ref[1:2, :] += q_acc


def _norm_body(y_ref, sc_ref, out_ref):
    z = sc_ref[0:1, :] * y_ref[0] + sc_ref[1:2, :]
    out_ref[0] = jnp.where(z >= 0.0, z, 0.2 * z)


def kernel(features, W, gamma, beta):
    x = jnp.squeeze(features, axis=-1)
    pts = jnp.transpose(x, (0, 2, 1))
    pts_flat = pts.reshape(R, C)
    sq = jnp.sum(pts * pts, axis=-1)
    sqr = sq[:, None, :]
    sqc = sq.reshape(R, 1)

    dmat = pl.pallas_call(
        _dist_body,
        grid=(R // TN,),
        in_specs=[
            pl.BlockSpec((1, N, C), lambda t: (t // (N // TN), 0, 0)),
            pl.BlockSpec((TN, C), lambda t: (t, 0)),
            pl.BlockSpec((1, 1, N), lambda t: (t // (N // TN), 0, 0)),
            pl.BlockSpec((TN, 1), lambda t: (t, 0)),
        ],
        out_specs=pl.BlockSpec((TN, N), lambda t: (t, 0)),
        out_shape=jax.ShapeDtypeStruct((R, N), jnp.float32),
    )(pts, pts_flat, sqr, sqc)

    mesh = plsc.VectorSubcoreMesh(core_axis_name="c", subcore_axis_name="s")
    candv, candi = pl.kernel(
        _sc_filter_body,
        mesh=mesh,
        compiler_params=pltpu.CompilerParams(needs_layout_passes=False),
        out_type=[
            jax.ShapeDtypeStruct((R * CAND,), jnp.float32),
            jax.ShapeDtypeStruct((R * CAND,), jnp.float32),
        ],
        scratch_types=[
            pltpu.VMEM((RB, N), jnp.float32),
            pltpu.VMEM((RB * CPAD,), jnp.float32),
            pltpu.VMEM((RB * CAND,), jnp.float32),
            pltpu.VMEM((RB * CAND,), jnp.float32),
        ],
    )(dmat)

    gidx, dist = pl.pallas_call(
        _select_body,
        grid=(B, N // TS),
        in_specs=[
            pl.BlockSpec((1, 1, TS, CAND), lambda b, t: (b, t, 0, 0)),
            pl.BlockSpec((1, 1, TS, CAND), lambda b, t: (b, t, 0, 0)),
        ],
        out_specs=[
            pl.BlockSpec((1, 1, TS, K), lambda b, t: (b, t, 0, 0)),
            pl.BlockSpec((1, 1, TS, K), lambda b, t: (b, t, 0, 0)),
        ],
        out_shape=[
            jax.ShapeDtypeStruct((B, N // TS, TS, K), jnp.int32),
            jax.ShapeDtypeStruct((B, N // TS, TS, K), jnp.float32),
        ],
    )(candv.reshape(B, N // TS, TS, CAND),
      candi.reshape(B, N // TS, TS, CAND))
    gidx_flat = gidx.reshape(R * K)
    dist_flat = dist.reshape(R, K)

    feat = pl.kernel(
        _sc_gather_body,
        mesh=mesh,
        compiler_params=pltpu.CompilerParams(needs_layout_passes=False),
        out_type=jax.ShapeDtypeStruct((R * K * C,), jnp.float32),
        scratch_types=[
            pltpu.VMEM((N * C,), jnp.float32),
            pltpu.VMEM((GCH,), jnp.int32),
            pltpu.VMEM((GCH * C,), jnp.float32),
        ],
    )(gidx_flat, pts_flat.reshape(R * C))

    y, sums = pl.pallas_call(
        _conv_body,
        grid=(R // TS,),
        in_specs=[
            pl.BlockSpec((TS, C), lambda t: (t, 0)),
            pl.BlockSpec((TS, K * C), lambda t: (t, 0)),
            pl.BlockSpec((TS, K), lambda t: (t, 0)),
            pl.BlockSpec((D_OUT, 4 * C), lambda t: (0, 0)),
        ],
        out_specs=[
            pl.BlockSpec((K, TS, D_OUT), lambda t: (0, t, 0)),
            pl.BlockSpec((2, D_OUT), lambda t: (0, 0)),
        ],
        out_shape=[
            jax.ShapeDtypeStruct((K, R, D_OUT), jnp.float32),
            jax.ShapeDtypeStruct((2, D_OUT), jnp.float32),
        ],
    )(pts_flat, feat.reshape(R, K * C), dist_flat, W)

    cnt = jnp.float32(R * K)
    mean = sums[0] / cnt
    var = sums[1] / cnt - mean * mean
    scale = gamma / jnp.sqrt(var + EPS)
    bias = beta - scale * mean
    sb = jnp.stack([scale, bias], axis=0)

    z = pl.pallas_call(
        _norm_body,
        grid=(K, R // 1024),
        in_specs=[
            pl.BlockSpec((1, 1024, D_OUT), lambda j, t: (j, t, 0)),
            pl.BlockSpec((2, D_OUT), lambda j, t: (0, 0)),
        ],
        out_specs=pl.BlockSpec((1, 1024, D_OUT), lambda j, t: (j, t, 0)),
        out_shape=jax.ShapeDtypeStruct((K, R, D_OUT), jnp.float32),
    )(y, sb)

    return z.reshape(K, B, N, D_OUT).transpose(1, 3, 2, 0)


# speedup vs baseline: 7.3437x; 1.0302x over previous
import functools

import jax
import jax.numpy as jnp
from jax import lax
from jax.experimental import pallas as pl
from jax.experimental.pallas import tpu as pltpu
from jax.experimental.pallas import tpu_sc as plsc

B, C, N, K = 8, 16, 2048, 20
D_OUT = 64
EPS = 1e-5
BIG = 3.0e38

R = B * N
NW = 32
RPW = R // NW
RB = 16
RG = 8
NBLK = RPW // RB
CAND = 64
CPAD = 96
NCH = N // 16

TN = 256
TS = 512
GPW = (R * K) // NW
GCH = 512


def _dist_body(ptsb_ref, xt_ref, sqr_ref, sqc_ref, d_ref):
    pts = ptsb_ref[0]
    xt = xt_ref[...]
    prod = jax.lax.dot_general(
        xt, pts, (((1,), (1,)), ((), ())),
        preferred_element_type=jnp.float32)
    d_ref[...] = (sqc_ref[...] + sqr_ref[0]) - 2.0 * prod


def _sc_filter_body(dmat, candv_hbm, candi_hbm, buf, oi, pkv, pki):
    wid = lax.axis_index("s") * 2 + lax.axis_index("c")
    iota = lax.iota(jnp.int32, 16)
    iota_f = iota.astype(jnp.float32)
    bigv = jnp.full((16,), BIG, jnp.float32)

    for q in range(RB * CPAD // 16):
        oi[pl.ds(q * 16, 16)] = jnp.zeros((16,), jnp.float32)

    def block(blk, _):
        row0 = wid * RPW + blk * RB
        pltpu.sync_copy(dmat.at[pl.ds(row0, RB), :], buf)

        for g0 in range(0, RB, RG):
            def p1(c, carry):
                out = []
                for r in range(RG):
                    m1, m2 = carry[2 * r], carry[2 * r + 1]
                    v = buf[g0 + r, pl.ds(c * 16, 16)]
                    a = jnp.minimum(m1, v)
                    bmx = jnp.maximum(m1, v)
                    out += [a, jnp.minimum(m2, bmx)]
                return tuple(out)

            accs = lax.fori_loop(0, NCH, p1, (bigv,) * (2 * RG), unroll=4)

            ts = []
            for r in range(RG):
                m1, m2 = accs[2 * r], accs[2 * r + 1]
                s1, _ = plsc.sort_key_val(m1, m1)
                s2, _ = plsc.sort_key_val(m2, m2)
                hi = jnp.maximum(s1, lax.rev(s2, (0,)))
                shi, _ = plsc.sort_key_val(hi, hi)
                ts.append(jnp.min(jnp.where(iota == 3, shi, BIG)))

            def flt(c, cnts):
                new = []
                idxf = (c * 16).astype(jnp.float32) + iota_f
                for r in range(RG):
                    v = buf[g0 + r, pl.ds(c * 16, 16)]
                    mask = v <= ts[r]
                    off = (g0 + r) * CPAD + jnp.minimum(cnts[r], CAND)
                    plsc.store_compressed(oi.at[pl.ds(off, 16)], idxf,
                                          mask=mask)
                    new.append(cnts[r] + jnp.sum(mask.astype(jnp.int32)))
                return tuple(new)

            cnts = lax.fori_loop(0, NCH, flt, (jnp.int32(0),) * RG, unroll=2)

            for r in range(RG):
                rsplat = jnp.full((16,), g0 + r, jnp.int32)
                for q in range(CAND // 16):
                    lane = q * 16 + iota
                    idxf = oi[pl.ds((g0 + r) * CPAD + q * 16, 16)]
                    vals = plsc.load_gather(
                        buf, [rsplat, idxf.astype(jnp.int32)])
                    vals = jnp.where(lane < cnts[r], vals, bigv)
                    pkv[pl.ds((g0 + r) * CAND + q * 16, 16)] = vals
                    pki[pl.ds((g0 + r) * CAND + q * 16, 16)] = idxf

        pltpu.sync_copy(pkv, candv_hbm.at[pl.ds(row0 * CAND, RB * CAND)])
        pltpu.sync_copy(pki, candi_hbm.at[pl.ds(row0 * CAND, RB * CAND)])
        return 0

    lax.fori_loop(0, NBLK, block, 0)


def _select_body(cv_ref, ci_ref, gidx_ref, dist_ref):
    cv = cv_ref[0, 0]
    ci = ci_ref[0, 0]
    col = jax.lax.broadcasted_iota(jnp.int32, (TS, CAND), 1)
    gs, ms = [], []
    for j in range(K):
        m = jnp.min(cv, axis=1, keepdims=True)
        sel = jnp.min(jnp.where(cv == m, col, CAND), axis=1, keepdims=True)
        onehot = col == sel
        gi = jnp.min(jnp.where(onehot, ci, BIG), axis=1, keepdims=True)
        gs.append(gi)
        ms.append(m)
        cv = jnp.where(onehot, BIG, cv)
    gidx = jnp.concatenate(gs, axis=1)
    gidx_ref[0, 0] = gidx.astype(jnp.int32)
    dist_ref[0, 0] = jnp.concatenate(ms, axis=1)


def _sc_gather_body(gidx_hbm, pts_hbm, feat_hbm, ptbuf, idx_v, out_v):
    wid = lax.axis_index("s") * 2 + lax.axis_index("c")
    b = wid // (NW // B)
    pltpu.sync_copy(pts_hbm.at[pl.ds(b * (N * C), N * C)], ptbuf)
    iota = lax.iota(jnp.int32, 16)

    def chunk(i, _):
        c0 = wid * GPW + i * GCH
        pltpu.sync_copy(gidx_hbm.at[pl.ds(c0, GCH)], idx_v)
        for g in range(GCH // 16):
            base = idx_v[pl.ds(g * 16, 16)] * C
            dbase = g * (16 * C) + iota * C
            for c in range(C):
                vals = plsc.load_gather(ptbuf, [base + c])
                plsc.store_scatter(out_v, [dbase + c], vals)
        pltpu.sync_copy(out_v, feat_hbm.at[pl.ds(c0 * C, GCH * C)])
        return 0

    lax.fori_loop(0, GPW // GCH, chunk, 0)


def _conv_body(xt_ref, feat_ref, dist_ref, w_ref, y_ref, sums_ref):
    t = pl.program_id(0)
    xt = xt_ref[...]
    w = w_ref[...]
    s_acc = jnp.zeros((1, D_OUT), jnp.float32)
    q_acc = jnp.zeros((1, D_OUT), jnp.float32)
    for j in range(K):
        feat = feat_ref[:, j * C:(j + 1) * C]
        m = dist_ref[:, j:j + 1]
        f = jnp.concatenate(
            [xt, feat, feat - xt, jnp.broadcast_to(m, (TS, C))], axis=1)
        y = jax.lax.dot_general(
            f, w, (((1,), (1,)), ((), ())),
            preferred_element_type=jnp.float32)
        y_ref[j] = y
        s_acc = s_acc + jnp.sum(y, axis=0, keepdims=True)
        q_acc = q_acc + jnp.sum(y * y, axis=0, keepdims=True)

    @pl.when(t == 0)
    def _():
        sums_ref[...] = jnp.zeros_like(sums_ref)

    sums_ref[0:1, :] += s_acc
    sums_ref[1:2, :] += q_acc


def _norm_body(y_ref, sc_ref, out_ref):
    z = sc_ref[0:1, :] * y_ref[0] + sc_ref[1:2, :]
    out_ref[0] = jnp.where(z >= 0.0, z, 0.2 * z)


def kernel(features, W, gamma, beta):
    x = jnp.squeeze(features, axis=-1)
    pts = jnp.transpose(x, (0, 2, 1))
    pts_flat = pts.reshape(R, C)
    sq = jnp.sum(pts * pts, axis=-1)
    sqr = sq[:, None, :]
    sqc = sq.reshape(R, 1)

    dmat = pl.pallas_call(
        _dist_body,
        grid=(R // TN,),
        in_specs=[
            pl.BlockSpec((1, N, C), lambda t: (t // (N // TN), 0, 0)),
            pl.BlockSpec((TN, C), lambda t: (t, 0)),
            pl.BlockSpec((1, 1, N), lambda t: (t // (N // TN), 0, 0)),
            pl.BlockSpec((TN, 1), lambda t: (t, 0)),
        ],
        out_specs=pl.BlockSpec((TN, N), lambda t: (t, 0)),
        out_shape=jax.ShapeDtypeStruct((R, N), jnp.float32),
    )(pts, pts_flat, sqr, sqc)

    mesh = plsc.VectorSubcoreMesh(core_axis_name="c", subcore_axis_name="s")
    candv, candi = pl.kernel(
        _sc_filter_body,
        mesh=mesh,
        compiler_params=pltpu.CompilerParams(needs_layout_passes=False),
        out_type=[
            jax.ShapeDtypeStruct((R * CAND,), jnp.float32),
            jax.ShapeDtypeStruct((R * CAND,), jnp.float32),
        ],
        scratch_types=[
            pltpu.VMEM((RB, N), jnp.float32),
            pltpu.VMEM((RB * CPAD,), jnp.float32),
            pltpu.VMEM((RB * CAND,), jnp.float32),
            pltpu.VMEM((RB * CAND,), jnp.float32),
        ],
    )(dmat)

    gidx, dist = pl.pallas_call(
        _select_body,
        grid=(B, N // TS),
        in_specs=[
            pl.BlockSpec((1, 1, TS, CAND), lambda b, t: (b, t, 0, 0)),
            pl.BlockSpec((1, 1, TS, CAND), lambda b, t: (b, t, 0, 0)),
        ],
        out_specs=[
            pl.BlockSpec((1, 1, TS, K), lambda b, t: (b, t, 0, 0)),
            pl.BlockSpec((1, 1, TS, K), lambda b, t: (b, t, 0, 0)),
        ],
        out_shape=[
            jax.ShapeDtypeStruct((B, N // TS, TS, K), jnp.int32),
            jax.ShapeDtypeStruct((B, N // TS, TS, K), jnp.float32),
        ],
    )(candv.reshape(B, N // TS, TS, CAND),
      candi.reshape(B, N // TS, TS, CAND))
    gidx_flat = gidx.reshape(R * K)
    dist_flat = dist.reshape(R, K)

    feat = pl.kernel(
        _sc_gather_body,
        mesh=mesh,
        compiler_params=pltpu.CompilerParams(needs_layout_passes=False),
        out_type=jax.ShapeDtypeStruct((R * K * C,), jnp.float32),
        scratch_types=[
            pltpu.VMEM((N * C,), jnp.float32),
            pltpu.VMEM((GCH,), jnp.int32),
            pltpu.VMEM((GCH * C,), jnp.float32),
        ],
    )(gidx_flat, pts_flat.reshape(R * C))

    y, sums = pl.pallas_call(
        _conv_body,
        grid=(R // TS,),
        in_specs=[
            pl.BlockSpec((TS, C), lambda t: (t, 0)),
            pl.BlockSpec((TS, K * C), lambda t: (t, 0)),
            pl.BlockSpec((TS, K), lambda t: (t, 0)),
            pl.BlockSpec((D_OUT, 4 * C), lambda t: (0, 0)),
        ],
        out_specs=[
            pl.BlockSpec((K, TS, D_OUT), lambda t: (0, t, 0)),
            pl.BlockSpec((2, D_OUT), lambda t: (0, 0)),
        ],
        out_shape=[
            jax.ShapeDtypeStruct((K, R, D_OUT), jnp.float32),
            jax.ShapeDtypeStruct((2, D_OUT), jnp.float32),
        ],
    )(pts_flat, feat.reshape(R, K * C), dist_flat, W)

    cnt = jnp.float32(R * K)
    mean = sums[0] / cnt
    var = sums[1] / cnt - mean * mean
    scale = gamma / jnp.sqrt(var + EPS)
    bias = beta - scale * mean
    sb = jnp.stack([scale, bias], axis=0)

    z = pl.pallas_call(
        _norm_body,
        grid=(K, R // 1024),
        in_specs=[
            pl.BlockSpec((1, 1024, D_OUT), lambda j, t: (j, t, 0)),
            pl.BlockSpec((2, D_OUT), lambda j, t: (0, 0)),
        ],
        out_specs=pl.BlockSpec((1, 1024, D_OUT), lambda j, t: (j, t, 0)),
        out_shape=jax.ShapeDtypeStruct((K, R, D_OUT), jnp.float32),
    )(y, sb)

    return z.reshape(K, B, N, D_OUT).transpose(1, 3, 2, 0)
